# Initial kernel scaffold; baseline (speedup 1.0000x reference)
import jax
import jax.numpy as jnp
from jax.experimental import pallas as pl


def kernel(x, edge_index, edge_attr, g_edge, v_edge, W_gcn, b_gcn, W1, b1, W2, b2):
    raise NotImplementedError("write your pallas kernel here")


# speedup vs baseline: 20.5304x; 20.5304x over previous
import functools

import jax
import jax.numpy as jnp
from jax import lax
from jax.experimental import pallas as pl
from jax.experimental.pallas import tpu as pltpu
from jax.experimental.pallas import tpu_sc as plsc

N_NODES = 10000
D_NODE = 128
D_LAT = 64
H_DIM = 512
N_OUT = 128
N_EDGES = 320000

NC, NS = 2, 16
NW = NC * NS
B = 128
NB = 80
EPW = NB * B
EPAD = NW * EPW
NPAD = 10240
RPT = NPAD // NS
RB = 1000
GRID = N_NODES // RB

_INTERPRET = False

_mesh = plsc.VectorSubcoreMesh(core_axis_name="c", subcore_axis_name="s",
                               num_cores=NC, num_subcores=NS)
_sc_params = pltpu.CompilerParams(needs_layout_passes=False,
                                  use_tc_tiling_on_sc=False)


@functools.partial(
    pl.kernel,
    out_type=jax.ShapeDtypeStruct((NW, NPAD), jnp.float32),
    mesh=_mesh,
    scratch_types=[
        pltpu.VMEM((NB, B), jnp.int32),
        pltpu.VMEM((NPAD,), jnp.float32),
    ],
    compiler_params=_sc_params,
    interpret=_INTERPRET,
)
def _deg_kernel(dst_hbm, zero_hbm, out_hbm, dst_v, hist_v):
    cid = lax.axis_index("c")
    sid = lax.axis_index("s")
    wid = cid * NS + sid
    pltpu.sync_copy(dst_hbm.at[wid], dst_v)
    pltpu.sync_copy(zero_hbm, hist_v)
    ones16 = jnp.ones((16,), jnp.float32)

    @pl.loop(0, NB)
    def _(b):
        @pl.loop(0, B, step=16)
        def _(k):
            idx = dst_v[b, pl.ds(k, 16)]
            plsc.addupdate_scatter(hist_v, [idx], ones16)

    pltpu.sync_copy(hist_v, out_hbm.at[wid])


@functools.partial(
    pl.kernel,
    out_type=jax.ShapeDtypeStruct((NC, NPAD, D_LAT), jnp.float32),
    mesh=_mesh,
    scratch_types=[
        pltpu.VMEM((NB, B), jnp.int32),
        pltpu.VMEM((NB, B), jnp.int32),
        pltpu.VMEM((B, D_LAT), jnp.float32),
        pltpu.VMEM_SHARED((NPAD, D_LAT), jnp.float32),
        pltpu.SemaphoreType.DMA,
    ],
    compiler_params=_sc_params,
    interpret=_INTERPRET,
)
def _scatter_kernel(src_hbm, dst_hbm, hs_hbm, zrows_hbm, out_hbm,
                    src_v, dst_v, buf, agg_sh, sem):
    cid = lax.axis_index("c")
    sid = lax.axis_index("s")
    wid = cid * NS + sid
    pltpu.sync_copy(src_hbm.at[wid], src_v)
    pltpu.sync_copy(dst_hbm.at[wid], dst_v)
    pltpu.sync_copy(zrows_hbm, agg_sh.at[pl.ds(sid * RPT, RPT)])
    plsc.subcore_barrier()

    @pl.loop(0, NB)
    def _(b):
        pltpu.async_copy(hs_hbm.at[src_v.at[b]], buf, sem).wait()
        pltpu.sync_copy(buf, agg_sh.at[dst_v.at[b]], add=True)

    plsc.subcore_barrier()
    pltpu.sync_copy(agg_sh.at[pl.ds(sid * RPT, RPT)],
                    out_hbm.at[cid, pl.ds(sid * RPT, RPT)])


def _mm_body(x_ref, w_ref, o_ref):
    o_ref[...] = jnp.dot(x_ref[...], w_ref[...],
                         preferred_element_type=jnp.float32)


_mm_call = pl.pallas_call(
    _mm_body,
    grid=(GRID,),
    in_specs=[
        pl.BlockSpec((RB, D_NODE), lambda i: (i, 0)),
        pl.BlockSpec((D_NODE, D_LAT), lambda i: (0, 0)),
    ],
    out_specs=pl.BlockSpec((RB, D_LAT), lambda i: (i, 0)),
    out_shape=jax.ShapeDtypeStruct((N_NODES, D_LAT), jnp.float32),
    interpret=_INTERPRET,
)


def _scale_body(h_ref, degT_ref, o_ref):
    deg = jnp.sum(degT_ref[...], axis=1, keepdims=True) + 1.0
    dinv = lax.rsqrt(deg)
    o_ref[...] = h_ref[...] * dinv


_scale_call = pl.pallas_call(
    _scale_body,
    grid=(GRID,),
    in_specs=[
        pl.BlockSpec((RB, D_LAT), lambda i: (i, 0)),
        pl.BlockSpec((RB, NW), lambda i: (i, 0)),
    ],
    out_specs=pl.BlockSpec((RB, D_LAT), lambda i: (i, 0)),
    out_shape=jax.ShapeDtypeStruct((N_NODES, D_LAT), jnp.float32),
    interpret=_INTERPRET,
)


def _mlp_body(aggp_ref, hs_ref, degT_ref, bgcn_ref, W1_ref, b1_ref,
              W2_ref, b2_ref, o_ref):
    deg = jnp.sum(degT_ref[...], axis=1, keepdims=True) + 1.0
    dinv = lax.rsqrt(deg)
    agg = aggp_ref[0] + aggp_ref[1]
    g = dinv * (agg + hs_ref[...]) + bgcn_ref[...]
    W1s = W1_ref[pl.ds(0, D_LAT), :]
    for k in range(1, H_DIM // D_LAT):
        W1s = W1s + W1_ref[pl.ds(k * D_LAT, D_LAT), :]
    hh = jnp.dot(g, W1s, preferred_element_type=jnp.float32) + b1_ref[...]
    hh = jnp.maximum(hh, 0.0)
    o_ref[...] = jnp.dot(hh, W2_ref[...],
                         preferred_element_type=jnp.float32) + b2_ref[...]


_mlp_call = pl.pallas_call(
    _mlp_body,
    grid=(GRID,),
    in_specs=[
        pl.BlockSpec((NC, RB, D_LAT), lambda i: (0, i, 0)),
        pl.BlockSpec((RB, D_LAT), lambda i: (i, 0)),
        pl.BlockSpec((RB, NW), lambda i: (i, 0)),
        pl.BlockSpec((1, D_LAT), lambda i: (0, 0)),
        pl.BlockSpec((H_DIM, H_DIM), lambda i: (0, 0)),
        pl.BlockSpec((1, H_DIM), lambda i: (0, 0)),
        pl.BlockSpec((H_DIM, N_OUT), lambda i: (0, 0)),
        pl.BlockSpec((1, N_OUT), lambda i: (0, 0)),
    ],
    out_specs=pl.BlockSpec((RB, N_OUT), lambda i: (i, 0)),
    out_shape=jax.ShapeDtypeStruct((N_NODES, N_OUT), jnp.float32),
    interpret=_INTERPRET,
)


def kernel(x, edge_index, edge_attr, g_edge, v_edge, W_gcn, b_gcn,
           W1, b1, W2, b2):
    src = edge_index[0].astype(jnp.int32)
    dst = edge_index[1].astype(jnp.int32)
    pad = EPAD - N_EDGES
    src_r = jnp.concatenate(
        [src, jnp.zeros((pad,), jnp.int32)]).reshape(NW, NB, B)
    dst_r = jnp.concatenate(
        [dst, jnp.full((pad,), N_NODES, jnp.int32)]).reshape(NW, NB, B)
    zhist = jnp.zeros((NPAD,), jnp.float32)
    zrows = jnp.zeros((RPT, D_LAT), jnp.float32)

    degp = _deg_kernel(dst_r, zhist)
    degT = degp.T
    h = _mm_call(x, W_gcn)
    hs = _scale_call(h, degT)
    aggp = _scatter_kernel(src_r, dst_r, hs, zrows)
    y = _mlp_call(aggp, hs, degT,
                  b_gcn.reshape(1, D_LAT), W1,
                  b1.reshape(1, H_DIM), W2, b2.reshape(1, N_OUT))
    return y


# speedup vs baseline: 23.5741x; 1.1483x over previous
import functools

import jax
import jax.numpy as jnp
from jax import lax
from jax.experimental import pallas as pl
from jax.experimental.pallas import tpu as pltpu
from jax.experimental.pallas import tpu_sc as plsc

N_NODES = 10000
D_NODE = 128
D_LAT = 64
H_DIM = 512
N_OUT = 128
N_EDGES = 320000

NC, NS = 2, 16
NW = NC * NS
B = 128
NB = 80
EPW = NB * B
EPAD = NW * EPW
NPAD = 10240
RPT = NPAD // NS
RB = 1000
GRID = N_NODES // RB

_INTERPRET = False

_mesh = plsc.VectorSubcoreMesh(core_axis_name="c", subcore_axis_name="s",
                               num_cores=NC, num_subcores=NS)
_sc_params = pltpu.CompilerParams(needs_layout_passes=False,
                                  use_tc_tiling_on_sc=False)


@functools.partial(
    pl.kernel,
    out_type=jax.ShapeDtypeStruct((NW, NPAD), jnp.float32),
    mesh=_mesh,
    scratch_types=[
        pltpu.VMEM((NB, B), jnp.int32),
        pltpu.VMEM((NPAD,), jnp.float32),
    ],
    compiler_params=_sc_params,
    interpret=_INTERPRET,
)
def _deg_kernel(dst_hbm, zero_hbm, out_hbm, dst_v, hist_v):
    cid = lax.axis_index("c")
    sid = lax.axis_index("s")
    wid = cid * NS + sid
    pltpu.sync_copy(dst_hbm.at[wid], dst_v)
    pltpu.sync_copy(zero_hbm, hist_v)
    ones16 = jnp.ones((16,), jnp.float32)

    @pl.loop(0, NB)
    def _(b):
        @pl.loop(0, B, step=16)
        def _(k):
            idx = dst_v[b, pl.ds(k, 16)]
            plsc.addupdate_scatter(hist_v, [idx], ones16)

    pltpu.sync_copy(hist_v, out_hbm.at[wid])


@functools.partial(
    pl.kernel,
    out_type=jax.ShapeDtypeStruct((NC, NPAD, D_LAT), jnp.float32),
    mesh=_mesh,
    scratch_types=[
        pltpu.VMEM((NB, B), jnp.int32),
        pltpu.VMEM((NB, B), jnp.int32),
        pltpu.VMEM((B, D_LAT), jnp.float32),
        pltpu.VMEM((B, D_LAT), jnp.float32),
        pltpu.VMEM((B, D_LAT), jnp.float32),
        pltpu.VMEM((B, D_LAT), jnp.float32),
        pltpu.VMEM_SHARED((NPAD, D_LAT), jnp.float32),
        pltpu.SemaphoreType.DMA,
        pltpu.SemaphoreType.DMA,
        pltpu.SemaphoreType.DMA,
        pltpu.SemaphoreType.DMA,
    ],
    compiler_params=_sc_params,
    interpret=_INTERPRET,
)
def _scatter_kernel(src_hbm, dst_hbm, hs_hbm, zrows_hbm, out_hbm,
                    src_v, dst_v, buf0, buf1, buf2, buf3, agg_sh,
                    sem0, sem1, sem2, sem3):
    cid = lax.axis_index("c")
    sid = lax.axis_index("s")
    wid = cid * NS + sid
    bufs = (buf0, buf1, buf2, buf3)
    sems = (sem0, sem1, sem2, sem3)
    nbuf = len(bufs)
    pltpu.sync_copy(src_hbm.at[wid], src_v)
    pltpu.sync_copy(dst_hbm.at[wid], dst_v)
    pltpu.sync_copy(zrows_hbm, agg_sh.at[pl.ds(sid * RPT, RPT)])
    plsc.subcore_barrier()

    for j in range(nbuf):
        pltpu.async_copy(hs_hbm.at[src_v.at[j]], bufs[j], sems[j])

    @pl.loop(0, NB, step=nbuf)
    def _(b):
        for j in range(nbuf):
            bj = b + j
            pltpu.make_async_copy(
                hs_hbm.at[src_v.at[bj]], bufs[j], sems[j]).wait()
            pltpu.sync_copy(bufs[j], agg_sh.at[dst_v.at[bj]], add=True)

            @pl.when(bj + nbuf < NB)
            def _():
                pltpu.async_copy(
                    hs_hbm.at[src_v.at[bj + nbuf]], bufs[j], sems[j])

    plsc.subcore_barrier()
    pltpu.sync_copy(agg_sh.at[pl.ds(sid * RPT, RPT)],
                    out_hbm.at[cid, pl.ds(sid * RPT, RPT)])


def _mm_body(x_ref, w_ref, o_ref):
    o_ref[...] = jnp.dot(x_ref[...], w_ref[...],
                         preferred_element_type=jnp.float32)


_mm_call = pl.pallas_call(
    _mm_body,
    grid=(GRID,),
    in_specs=[
        pl.BlockSpec((RB, D_NODE), lambda i: (i, 0)),
        pl.BlockSpec((D_NODE, D_LAT), lambda i: (0, 0)),
    ],
    out_specs=pl.BlockSpec((RB, D_LAT), lambda i: (i, 0)),
    out_shape=jax.ShapeDtypeStruct((N_NODES, D_LAT), jnp.float32),
    interpret=_INTERPRET,
)


def _scale_body(h_ref, degT_ref, o_ref):
    deg = jnp.sum(degT_ref[...], axis=1, keepdims=True) + 1.0
    dinv = lax.rsqrt(deg)
    o_ref[...] = h_ref[...] * dinv


_scale_call = pl.pallas_call(
    _scale_body,
    grid=(GRID,),
    in_specs=[
        pl.BlockSpec((RB, D_LAT), lambda i: (i, 0)),
        pl.BlockSpec((RB, NW), lambda i: (i, 0)),
    ],
    out_specs=pl.BlockSpec((RB, D_LAT), lambda i: (i, 0)),
    out_shape=jax.ShapeDtypeStruct((N_NODES, D_LAT), jnp.float32),
    interpret=_INTERPRET,
)


def _mlp_body(aggp_ref, hs_ref, degT_ref, bgcn_ref, W1_ref, b1_ref,
              W2_ref, b2_ref, o_ref):
    deg = jnp.sum(degT_ref[...], axis=1, keepdims=True) + 1.0
    dinv = lax.rsqrt(deg)
    agg = aggp_ref[0] + aggp_ref[1]
    g = dinv * (agg + hs_ref[...]) + bgcn_ref[...]
    W1s = W1_ref[pl.ds(0, D_LAT), :]
    for k in range(1, H_DIM // D_LAT):
        W1s = W1s + W1_ref[pl.ds(k * D_LAT, D_LAT), :]
    hh = jnp.dot(g, W1s, preferred_element_type=jnp.float32) + b1_ref[...]
    hh = jnp.maximum(hh, 0.0)
    o_ref[...] = jnp.dot(hh, W2_ref[...],
                         preferred_element_type=jnp.float32) + b2_ref[...]


_mlp_call = pl.pallas_call(
    _mlp_body,
    grid=(GRID,),
    in_specs=[
        pl.BlockSpec((NC, RB, D_LAT), lambda i: (0, i, 0)),
        pl.BlockSpec((RB, D_LAT), lambda i: (i, 0)),
        pl.BlockSpec((RB, NW), lambda i: (i, 0)),
        pl.BlockSpec((1, D_LAT), lambda i: (0, 0)),
        pl.BlockSpec((H_DIM, H_DIM), lambda i: (0, 0)),
        pl.BlockSpec((1, H_DIM), lambda i: (0, 0)),
        pl.BlockSpec((H_DIM, N_OUT), lambda i: (0, 0)),
        pl.BlockSpec((1, N_OUT), lambda i: (0, 0)),
    ],
    out_specs=pl.BlockSpec((RB, N_OUT), lambda i: (i, 0)),
    out_shape=jax.ShapeDtypeStruct((N_NODES, N_OUT), jnp.float32),
    interpret=_INTERPRET,
)


def kernel(x, edge_index, edge_attr, g_edge, v_edge, W_gcn, b_gcn,
           W1, b1, W2, b2):
    src = edge_index[0].astype(jnp.int32)
    dst = edge_index[1].astype(jnp.int32)
    pad = EPAD - N_EDGES
    src_r = jnp.concatenate(
        [src, jnp.zeros((pad,), jnp.int32)]).reshape(NW, NB, B)
    dst_r = jnp.concatenate(
        [dst, jnp.full((pad,), N_NODES, jnp.int32)]).reshape(NW, NB, B)
    zhist = jnp.zeros((NPAD,), jnp.float32)
    zrows = jnp.zeros((RPT, D_LAT), jnp.float32)

    degp = _deg_kernel(dst_r, zhist)
    degT = degp.T
    h = _mm_call(x, W_gcn)
    hs = _scale_call(h, degT)
    aggp = _scatter_kernel(src_r, dst_r, hs, zrows)
    y = _mlp_call(aggp, hs, degT,
                  b_gcn.reshape(1, D_LAT), W1,
                  b1.reshape(1, H_DIM), W2, b2.reshape(1, N_OUT))
    return y


# speedup vs baseline: 23.7266x; 1.0065x over previous
import functools

import jax
import jax.numpy as jnp
from jax import lax
from jax.experimental import pallas as pl
from jax.experimental.pallas import tpu as pltpu
from jax.experimental.pallas import tpu_sc as plsc

N_NODES = 10000
D_NODE = 128
D_LAT = 64
H_DIM = 512
N_OUT = 128
N_EDGES = 320000

NC, NS = 2, 16
NW = NC * NS
B = 128
NB = 80
EPW = NB * B
EPAD = NW * EPW
NPAD = 10240
RPT = NPAD // NS
RB = 1000
GRID = N_NODES // RB

_INTERPRET = False

_mesh = plsc.VectorSubcoreMesh(core_axis_name="c", subcore_axis_name="s",
                               num_cores=NC, num_subcores=NS)
_sc_params = pltpu.CompilerParams(needs_layout_passes=False,
                                  use_tc_tiling_on_sc=False)


@functools.partial(
    pl.kernel,
    out_type=jax.ShapeDtypeStruct((NW, NPAD), jnp.float32),
    mesh=_mesh,
    scratch_types=[
        pltpu.VMEM((NB, B), jnp.int32),
        pltpu.VMEM((NPAD,), jnp.float32),
    ],
    compiler_params=_sc_params,
    interpret=_INTERPRET,
)
def _deg_kernel(dst_hbm, zero_hbm, out_hbm, dst_v, hist_v):
    cid = lax.axis_index("c")
    sid = lax.axis_index("s")
    wid = cid * NS + sid
    pltpu.sync_copy(dst_hbm.at[wid], dst_v)
    pltpu.sync_copy(zero_hbm, hist_v)
    ones16 = jnp.ones((16,), jnp.float32)

    @pl.loop(0, NB)
    def _(b):
        @pl.loop(0, B, step=16)
        def _(k):
            idx = dst_v[b, pl.ds(k, 16)]
            plsc.addupdate_scatter(hist_v, [idx], ones16)

    pltpu.sync_copy(hist_v, out_hbm.at[wid])


@functools.partial(
    pl.kernel,
    out_type=jax.ShapeDtypeStruct((NC, NPAD, D_LAT), jnp.float32),
    mesh=_mesh,
    scratch_types=[
        pltpu.VMEM((NB, B), jnp.int32),
        pltpu.VMEM((NB, B), jnp.int32),
    ] + [pltpu.VMEM((B, D_LAT), jnp.float32)] * 8 + [
        pltpu.VMEM_SHARED((NPAD, D_LAT), jnp.float32),
    ] + [pltpu.SemaphoreType.DMA] * 16,
    compiler_params=_sc_params,
    interpret=_INTERPRET,
)
def _scatter_kernel(src_hbm, dst_hbm, hs_hbm, zrows_hbm, out_hbm,
                    src_v, dst_v, *rest):
    bufs = rest[:8]
    agg_sh = rest[8]
    gsems = rest[9:17]
    ssems = rest[17:25]
    nbuf = 8
    lead = nbuf // 2
    cid = lax.axis_index("c")
    sid = lax.axis_index("s")
    wid = cid * NS + sid
    pltpu.sync_copy(src_hbm.at[wid], src_v)
    pltpu.sync_copy(dst_hbm.at[wid], dst_v)
    pltpu.sync_copy(zrows_hbm, agg_sh.at[pl.ds(sid * RPT, RPT)])
    plsc.subcore_barrier()

    for j in range(lead):
        pltpu.async_copy(hs_hbm.at[src_v.at[j]], bufs[j], gsems[j])

    @pl.loop(0, NB, step=nbuf)
    def _(b):
        for j in range(nbuf):
            bj = b + j
            pltpu.make_async_copy(
                hs_hbm.at[src_v.at[bj]], bufs[j], gsems[j]).wait()
            pltpu.async_copy(bufs[j], agg_sh.at[dst_v.at[bj]],
                             ssems[j], add=True)
            jp = (j - lead) % nbuf

            @pl.when(bj >= lead)
            def _():
                pltpu.make_async_copy(
                    bufs[jp], agg_sh.at[dst_v.at[bj - lead]],
                    ssems[jp]).wait()

            @pl.when(bj + lead < NB)
            def _():
                pltpu.async_copy(
                    hs_hbm.at[src_v.at[bj + lead]], bufs[jp], gsems[jp])

    for j in range(lead):
        bj = NB - lead + j
        pltpu.make_async_copy(
            bufs[bj % nbuf], agg_sh.at[dst_v.at[bj]],
            ssems[bj % nbuf]).wait()

    plsc.subcore_barrier()
    pltpu.sync_copy(agg_sh.at[pl.ds(sid * RPT, RPT)],
                    out_hbm.at[cid, pl.ds(sid * RPT, RPT)])


def _mm_body(x_ref, w_ref, o_ref):
    o_ref[...] = jnp.dot(x_ref[...], w_ref[...],
                         preferred_element_type=jnp.float32)


_mm_call = pl.pallas_call(
    _mm_body,
    grid=(GRID,),
    in_specs=[
        pl.BlockSpec((RB, D_NODE), lambda i: (i, 0)),
        pl.BlockSpec((D_NODE, D_LAT), lambda i: (0, 0)),
    ],
    out_specs=pl.BlockSpec((RB, D_LAT), lambda i: (i, 0)),
    out_shape=jax.ShapeDtypeStruct((N_NODES, D_LAT), jnp.float32),
    interpret=_INTERPRET,
)


def _scale_body(h_ref, degT_ref, o_ref):
    deg = jnp.sum(degT_ref[...], axis=1, keepdims=True) + 1.0
    dinv = lax.rsqrt(deg)
    o_ref[...] = h_ref[...] * dinv


_scale_call = pl.pallas_call(
    _scale_body,
    grid=(GRID,),
    in_specs=[
        pl.BlockSpec((RB, D_LAT), lambda i: (i, 0)),
        pl.BlockSpec((RB, NW), lambda i: (i, 0)),
    ],
    out_specs=pl.BlockSpec((RB, D_LAT), lambda i: (i, 0)),
    out_shape=jax.ShapeDtypeStruct((N_NODES, D_LAT), jnp.float32),
    interpret=_INTERPRET,
)


def _mlp_body(aggp_ref, hs_ref, degT_ref, bgcn_ref, W1_ref, b1_ref,
              W2_ref, b2_ref, o_ref):
    deg = jnp.sum(degT_ref[...], axis=1, keepdims=True) + 1.0
    dinv = lax.rsqrt(deg)
    agg = aggp_ref[0] + aggp_ref[1]
    g = dinv * (agg + hs_ref[...]) + bgcn_ref[...]
    W1s = W1_ref[pl.ds(0, D_LAT), :]
    for k in range(1, H_DIM // D_LAT):
        W1s = W1s + W1_ref[pl.ds(k * D_LAT, D_LAT), :]
    hh = jnp.dot(g, W1s, preferred_element_type=jnp.float32) + b1_ref[...]
    hh = jnp.maximum(hh, 0.0)
    o_ref[...] = jnp.dot(hh, W2_ref[...],
                         preferred_element_type=jnp.float32) + b2_ref[...]


_mlp_call = pl.pallas_call(
    _mlp_body,
    grid=(GRID,),
    in_specs=[
        pl.BlockSpec((NC, RB, D_LAT), lambda i: (0, i, 0)),
        pl.BlockSpec((RB, D_LAT), lambda i: (i, 0)),
        pl.BlockSpec((RB, NW), lambda i: (i, 0)),
        pl.BlockSpec((1, D_LAT), lambda i: (0, 0)),
        pl.BlockSpec((H_DIM, H_DIM), lambda i: (0, 0)),
        pl.BlockSpec((1, H_DIM), lambda i: (0, 0)),
        pl.BlockSpec((H_DIM, N_OUT), lambda i: (0, 0)),
        pl.BlockSpec((1, N_OUT), lambda i: (0, 0)),
    ],
    out_specs=pl.BlockSpec((RB, N_OUT), lambda i: (i, 0)),
    out_shape=jax.ShapeDtypeStruct((N_NODES, N_OUT), jnp.float32),
    interpret=_INTERPRET,
)


def kernel(x, edge_index, edge_attr, g_edge, v_edge, W_gcn, b_gcn,
           W1, b1, W2, b2):
    src = edge_index[0].astype(jnp.int32)
    dst = edge_index[1].astype(jnp.int32)
    pad = EPAD - N_EDGES
    src_r = jnp.concatenate(
        [src, jnp.zeros((pad,), jnp.int32)]).reshape(NW, NB, B)
    dst_r = jnp.concatenate(
        [dst, jnp.full((pad,), N_NODES, jnp.int32)]).reshape(NW, NB, B)
    zhist = jnp.zeros((NPAD,), jnp.float32)
    zrows = jnp.zeros((RPT, D_LAT), jnp.float32)

    degp = _deg_kernel(dst_r, zhist)
    degT = degp.T
    h = _mm_call(x, W_gcn)
    hs = _scale_call(h, degT)
    aggp = _scatter_kernel(src_r, dst_r, hs, zrows)
    y = _mlp_call(aggp, hs, degT,
                  b_gcn.reshape(1, D_LAT), W1,
                  b1.reshape(1, H_DIM), W2, b2.reshape(1, N_OUT))
    return y


# speedup vs baseline: 25.4092x; 1.0709x over previous
import functools

import jax
import jax.numpy as jnp
from jax import lax
from jax.experimental import pallas as pl
from jax.experimental.pallas import tpu as pltpu
from jax.experimental.pallas import tpu_sc as plsc

N_NODES = 10000
D_NODE = 128
D_LAT = 64
H_DIM = 512
N_OUT = 128
N_EDGES = 320000

NC, NS = 2, 16
NW = NC * NS
B = 128
NB = 80
NBT = NC * NS * NB
EPAD = NBT * B
NB0 = 112
NB1 = 48
NBMAX = max(NB0, NB1)
NBUF = 4
LEAD = NBUF // 2
NPAD = 10240
RPT = NPAD // NS
RB = 1000
GRID = N_NODES // RB

_INTERPRET = False

_mesh = plsc.VectorSubcoreMesh(core_axis_name="c", subcore_axis_name="s",
                               num_cores=NC, num_subcores=NS)
_sc_params = pltpu.CompilerParams(needs_layout_passes=False,
                                  use_tc_tiling_on_sc=False)


@functools.partial(
    pl.kernel,
    out_type=jax.ShapeDtypeStruct((NW, NPAD), jnp.float32),
    mesh=_mesh,
    scratch_types=[
        pltpu.VMEM((NB, B), jnp.int32),
        pltpu.VMEM((NPAD,), jnp.float32),
    ],
    compiler_params=_sc_params,
    interpret=_INTERPRET,
)
def _deg_kernel(dst_hbm, zero_hbm, out_hbm, dst_v, hist_v):
    cid = lax.axis_index("c")
    sid = lax.axis_index("s")
    wid = cid * NS + sid
    pltpu.sync_copy(dst_hbm.at[pl.ds(wid * NB, NB)], dst_v)
    pltpu.sync_copy(zero_hbm, hist_v)
    ones16 = jnp.ones((16,), jnp.float32)

    @pl.loop(0, NB)
    def _(b):
        @pl.loop(0, B, step=16)
        def _(k):
            idx = dst_v[b, pl.ds(k, 16)]
            plsc.addupdate_scatter(hist_v, [idx], ones16)

    pltpu.sync_copy(hist_v, out_hbm.at[wid])


@functools.partial(
    pl.kernel,
    out_type=jax.ShapeDtypeStruct((NC, NPAD, D_LAT), jnp.float32),
    mesh=_mesh,
    scratch_types=[
        pltpu.VMEM((NBMAX, B), jnp.int32),
        pltpu.VMEM((NBMAX, B), jnp.int32),
    ] + [pltpu.VMEM((B, D_LAT), jnp.float32)] * NBUF + [
        pltpu.VMEM_SHARED((NPAD, D_LAT), jnp.float32),
    ] + [pltpu.SemaphoreType.DMA] * (2 * NBUF),
    compiler_params=_sc_params,
    interpret=_INTERPRET,
)
def _scatter_kernel(src_hbm, dst_hbm, hs_hbm, zrows_hbm, out_hbm,
                    src_v, dst_v, *rest):
    bufs = rest[:NBUF]
    agg_sh = rest[NBUF]
    gsems = rest[NBUF + 1:2 * NBUF + 1]
    ssems = rest[2 * NBUF + 1:3 * NBUF + 1]
    cid = lax.axis_index("c")
    sid = lax.axis_index("s")

    def edge_pipeline(nb, base):
        pltpu.sync_copy(src_hbm.at[pl.ds(base, nb)],
                        src_v.at[pl.ds(0, nb)])
        pltpu.sync_copy(dst_hbm.at[pl.ds(base, nb)],
                        dst_v.at[pl.ds(0, nb)])
        for j in range(LEAD):
            pltpu.async_copy(hs_hbm.at[src_v.at[j]], bufs[j], gsems[j])

        @pl.loop(0, nb, step=NBUF)
        def _(b):
            for j in range(NBUF):
                bj = b + j
                pltpu.make_async_copy(
                    hs_hbm.at[src_v.at[bj]], bufs[j], gsems[j]).wait()
                pltpu.async_copy(bufs[j], agg_sh.at[dst_v.at[bj]],
                                 ssems[j], add=True)
                jp = (j - LEAD) % NBUF

                @pl.when(bj >= LEAD)
                def _():
                    pltpu.make_async_copy(
                        bufs[jp], agg_sh.at[dst_v.at[bj - LEAD]],
                        ssems[jp]).wait()

                @pl.when(bj + LEAD < nb)
                def _():
                    pltpu.async_copy(
                        hs_hbm.at[src_v.at[bj + LEAD]], bufs[jp],
                        gsems[jp])

        for j in range(LEAD):
            bj = nb - LEAD + j
            pltpu.make_async_copy(
                bufs[bj % NBUF], agg_sh.at[dst_v.at[bj]],
                ssems[bj % NBUF]).wait()

    pltpu.sync_copy(zrows_hbm, agg_sh.at[pl.ds(sid * RPT, RPT)])
    plsc.subcore_barrier()

    @pl.when(cid == 0)
    def _():
        edge_pipeline(NB0, sid * NB0)

    @pl.when(cid == 1)
    def _():
        edge_pipeline(NB1, NS * NB0 + sid * NB1)

    plsc.subcore_barrier()
    pltpu.sync_copy(agg_sh.at[pl.ds(sid * RPT, RPT)],
                    out_hbm.at[cid, pl.ds(sid * RPT, RPT)])


def _mm_body(x_ref, w_ref, o_ref):
    o_ref[...] = jnp.dot(x_ref[...], w_ref[...],
                         preferred_element_type=jnp.float32)


_mm_call = pl.pallas_call(
    _mm_body,
    grid=(GRID,),
    in_specs=[
        pl.BlockSpec((RB, D_NODE), lambda i: (i, 0)),
        pl.BlockSpec((D_NODE, D_LAT), lambda i: (0, 0)),
    ],
    out_specs=pl.BlockSpec((RB, D_LAT), lambda i: (i, 0)),
    out_shape=jax.ShapeDtypeStruct((N_NODES, D_LAT), jnp.float32),
    interpret=_INTERPRET,
)


def _scale_body(h_ref, degT_ref, o_ref):
    deg = jnp.sum(degT_ref[...], axis=1, keepdims=True) + 1.0
    dinv = lax.rsqrt(deg)
    o_ref[...] = h_ref[...] * dinv


_scale_call = pl.pallas_call(
    _scale_body,
    grid=(GRID,),
    in_specs=[
        pl.BlockSpec((RB, D_LAT), lambda i: (i, 0)),
        pl.BlockSpec((RB, NW), lambda i: (i, 0)),
    ],
    out_specs=pl.BlockSpec((RB, D_LAT), lambda i: (i, 0)),
    out_shape=jax.ShapeDtypeStruct((N_NODES, D_LAT), jnp.float32),
    interpret=_INTERPRET,
)


def _mlp_body(aggp_ref, hs_ref, degT_ref, bgcn_ref, W1_ref, b1_ref,
              W2_ref, b2_ref, o_ref):
    deg = jnp.sum(degT_ref[...], axis=1, keepdims=True) + 1.0
    dinv = lax.rsqrt(deg)
    agg = aggp_ref[0] + aggp_ref[1]
    g = dinv * (agg + hs_ref[...]) + bgcn_ref[...]
    W1s = W1_ref[pl.ds(0, D_LAT), :]
    for k in range(1, H_DIM // D_LAT):
        W1s = W1s + W1_ref[pl.ds(k * D_LAT, D_LAT), :]
    hh = jnp.dot(g, W1s, preferred_element_type=jnp.float32) + b1_ref[...]
    hh = jnp.maximum(hh, 0.0)
    o_ref[...] = jnp.dot(hh, W2_ref[...],
                         preferred_element_type=jnp.float32) + b2_ref[...]


_mlp_call = pl.pallas_call(
    _mlp_body,
    grid=(GRID,),
    in_specs=[
        pl.BlockSpec((NC, RB, D_LAT), lambda i: (0, i, 0)),
        pl.BlockSpec((RB, D_LAT), lambda i: (i, 0)),
        pl.BlockSpec((RB, NW), lambda i: (i, 0)),
        pl.BlockSpec((1, D_LAT), lambda i: (0, 0)),
        pl.BlockSpec((H_DIM, H_DIM), lambda i: (0, 0)),
        pl.BlockSpec((1, H_DIM), lambda i: (0, 0)),
        pl.BlockSpec((H_DIM, N_OUT), lambda i: (0, 0)),
        pl.BlockSpec((1, N_OUT), lambda i: (0, 0)),
    ],
    out_specs=pl.BlockSpec((RB, N_OUT), lambda i: (i, 0)),
    out_shape=jax.ShapeDtypeStruct((N_NODES, N_OUT), jnp.float32),
    interpret=_INTERPRET,
)


def kernel(x, edge_index, edge_attr, g_edge, v_edge, W_gcn, b_gcn,
           W1, b1, W2, b2):
    src = edge_index[0].astype(jnp.int32)
    dst = edge_index[1].astype(jnp.int32)
    pad = EPAD - N_EDGES
    src_r = jnp.concatenate(
        [src, jnp.zeros((pad,), jnp.int32)]).reshape(NBT, B)
    dst_r = jnp.concatenate(
        [dst, jnp.full((pad,), N_NODES, jnp.int32)]).reshape(NBT, B)
    zhist = jnp.zeros((NPAD,), jnp.float32)
    zrows = jnp.zeros((RPT, D_LAT), jnp.float32)

    degp = _deg_kernel(dst_r, zhist)
    degT = degp.T
    h = _mm_call(x, W_gcn)
    hs = _scale_call(h, degT)
    aggp = _scatter_kernel(src_r, dst_r, hs, zrows)
    y = _mlp_call(aggp, hs, degT,
                  b_gcn.reshape(1, D_LAT), W1,
                  b1.reshape(1, H_DIM), W2, b2.reshape(1, N_OUT))
    return y
